# baseline (speedup 1.0000x reference)
import functools

import jax
import jax.numpy as jnp
from jax import lax
from jax.experimental import pallas as pl
from jax.experimental.pallas import tpu as pltpu
from jax.experimental.pallas import tpu_sc as plsc

N_NODES = 10000
N_EDGES = 320000
D = 128
DE = 16
NSEG = 128
NCORE = 2
NSUB = 16
NW = NCORE * NSUB
EPW = 10240
E_PAD = NW * EPW
CH = 128
NCH = EPW // CH
NBUF = 2
GPS = 2
SROWS = GPS * CH
NSLOT = NCH // GPS
EBLK = 8192


@functools.cache
def _mesh():
  return plsc.VectorSubcoreMesh(
      core_axis_name="c", subcore_axis_name="s",
      num_cores=NCORE, num_subcores=NSUB)


def _sc_gather_body(src_hbm, x_hbm, out, src_v, rows_v, sems, semw):
  c = lax.axis_index("c")
  s = lax.axis_index("s")
  wid = s * NCORE + c

  pltpu.sync_copy(src_hbm.at[wid], src_v)

  def gather_slot(p, slot):
    for j in range(GPS):
      pltpu.async_copy(x_hbm.at[src_v.at[GPS * p + j]],
                       rows_v.at[slot, pl.ds(j * CH, CH)], sems.at[slot])

  def wait_slot(slot):
    pltpu.make_async_copy(x_hbm.at[pl.ds(0, SROWS)], rows_v.at[slot],
                          sems.at[slot]).wait()

  def wb(p, slot):
    return pltpu.make_async_copy(
        rows_v.at[slot], out.at[pl.ds(wid * EPW + p * SROWS, SROWS)],
        semw.at[slot])

  for b in range(NBUF):
    gather_slot(b, b)

  def group(g, _):
    for b in range(NBUF):
      p = g * NBUF + b
      wait_slot(b)
      wb(p, b).start()

      @pl.when(p + NBUF < NSLOT)
      def _():
        wb(p, b).wait()
        gather_slot(p + NBUF, b)
    return 0
  lax.fori_loop(0, NSLOT // NBUF, group, 0)

  last = (NSLOT // NBUF - 1) * NBUF
  for b in range(NBUF):
    wb(last + b, b).wait()


def _sc_gather(srcp, x):
  f = pl.kernel(
      _sc_gather_body,
      out_type=jax.ShapeDtypeStruct((E_PAD, D), jnp.float32),
      mesh=_mesh(),
      scratch_types=[
          pltpu.VMEM((NCH, CH), jnp.int32),
          pltpu.VMEM((NBUF, SROWS, D), jnp.float32),
          pltpu.SemaphoreType.DMA((NBUF,)),
          pltpu.SemaphoreType.DMA((NBUF,)),
      ],
  )
  return f(srcp.reshape(NW, NCH, CH), x)


def _embed_body(xf, W, b, out):
  out[...] = jnp.maximum(
      jnp.dot(xf[...], W[...], preferred_element_type=jnp.float32) + b[...],
      0.0)


def _embed(edge_feat_pad, W_ef, b_ef):
  return pl.pallas_call(
      _embed_body,
      grid=(E_PAD // EBLK,),
      in_specs=[
          pl.BlockSpec((EBLK, DE), lambda i: (i, 0)),
          pl.BlockSpec((DE, DE), lambda i: (0, 0)),
          pl.BlockSpec((1, DE), lambda i: (0, 0)),
      ],
      out_specs=pl.BlockSpec((EBLK, DE), lambda i: (i, 0)),
      out_shape=jax.ShapeDtypeStruct((E_PAD, DE), jnp.float32),
  )(edge_feat_pad, W_ef, b_ef.reshape(1, DE))


def _edge_mm_body(xg, ef, W, b, out):
  m = jnp.concatenate([xg[...], ef[...]], axis=1)
  out[...] = jnp.dot(m, W[...], preferred_element_type=jnp.float32) + b[...]


def _edge_mm(ghx, ef, W, b):
  return pl.pallas_call(
      _edge_mm_body,
      grid=(E_PAD // EBLK,),
      in_specs=[
          pl.BlockSpec((EBLK, D), lambda i: (i, 0)),
          pl.BlockSpec((EBLK, DE), lambda i: (i, 0)),
          pl.BlockSpec((D + DE, D), lambda i: (0, 0)),
          pl.BlockSpec((1, D), lambda i: (0, 0)),
      ],
      out_specs=pl.BlockSpec((EBLK, D), lambda i: (i, 0)),
      out_shape=jax.ShapeDtypeStruct((E_PAD, D), jnp.float32),
  )(ghx, ef, W, b.reshape(1, D))


def _scatter_mean_x(vals, idx, num_segments):
  s = jax.ops.segment_sum(vals, idx, num_segments=num_segments)
  c = jax.ops.segment_sum(jnp.ones((vals.shape[0],), dtype=vals.dtype), idx,
                          num_segments=num_segments)
  c = jnp.clip(c, 1.0, None)[:, None]
  return s / c


def _graph_norm_x(x, batch, weight, bias, mean_scale):
  cnt = jax.ops.segment_sum(jnp.ones((x.shape[0],), dtype=x.dtype), batch,
                            num_segments=NSEG)
  cnt = jnp.clip(cnt, 1.0, None)[:, None]
  mean = jax.ops.segment_sum(x, batch, num_segments=NSEG) / cnt
  sub = x - mean[batch] * mean_scale
  var = jax.ops.segment_sum(sub * sub, batch, num_segments=NSEG) / cnt
  std = jnp.sqrt(var + 1e-5)
  return weight * sub / std[batch] + bias


def kernel(x, edge_index, edge_features, batch, W_ef, b_ef, W_c0, b_c0,
           W_c1, b_c1, W_c2, b_c2, gn_weight, gn_bias, gn_mean_scale,
           W_f1, b_f1, W_f2, b_f2):
  pad = E_PAD - N_EDGES
  srcp = jnp.concatenate([edge_index[0], jnp.zeros((pad,), jnp.int32)])
  dst = edge_index[1]
  efp = jnp.concatenate(
      [edge_features, jnp.zeros((pad, DE), jnp.float32)], axis=0)

  ef = _embed(efp, W_ef, b_ef)
  h = x
  for (W, b) in ((W_c0, b_c0), (W_c1, b_c1), (W_c2, b_c2)):
    ghx = _sc_gather(srcp, h)
    he = _edge_mm(ghx, ef, W, b)[:N_EDGES]
    h = _scatter_mean_x(he, dst, N_NODES)
    h = jax.nn.relu(h)
    h = _graph_norm_x(h, batch, gn_weight, gn_bias, gn_mean_scale)

  xout = _scatter_mean_x(h, batch, NSEG)
  xout = xout.reshape(-1).reshape((-1, 2 * D))
  xout1 = xout[:, :D]
  xout2 = xout[:, D:]
  xsub = xout1 - xout2
  scores = jax.nn.softmax(jax.nn.relu(xsub @ W_f1 + b_f1) @ W_f2 + b_f2,
                          axis=-1)
  return (scores.reshape(-1), h, xout1, xout2)


# speedup vs baseline: 1.0913x; 1.0913x over previous
import functools

import jax
import jax.numpy as jnp
from jax import lax
from jax.experimental import pallas as pl
from jax.experimental.pallas import tpu as pltpu
from jax.experimental.pallas import tpu_sc as plsc

N_NODES = 10000
N_EDGES = 320000
D = 128
DE = 16
NSEG = 128
NCORE = 2
NSUB = 16
NW = NCORE * NSUB
EPW = 10240
E_PAD = NW * EPW
CH = 128
NCH = EPW // CH
NBUF = 2
GPS = 2
SROWS = GPS * CH
NSLOT = NCH // GPS
EBLK = 8000


@functools.cache
def _mesh():
  return plsc.VectorSubcoreMesh(
      core_axis_name="c", subcore_axis_name="s",
      num_cores=NCORE, num_subcores=NSUB)


def _sc_gather_body(src_hbm, x_hbm, out, src_v, rows_v, sems, semw):
  c = lax.axis_index("c")
  s = lax.axis_index("s")
  wid = s * NCORE + c

  pltpu.sync_copy(src_hbm.at[wid], src_v)

  def gather_slot(p, slot):
    for j in range(GPS):
      pltpu.async_copy(x_hbm.at[src_v.at[GPS * p + j]],
                       rows_v.at[slot, pl.ds(j * CH, CH)], sems.at[slot])

  def wait_slot(slot):
    pltpu.make_async_copy(x_hbm.at[pl.ds(0, SROWS)], rows_v.at[slot],
                          sems.at[slot]).wait()

  def wb(p, slot):
    return pltpu.make_async_copy(
        rows_v.at[slot], out.at[pl.ds(wid * EPW + p * SROWS, SROWS)],
        semw.at[slot])

  for b in range(NBUF):
    gather_slot(b, b)

  def group(g, _):
    for b in range(NBUF):
      p = g * NBUF + b
      wait_slot(b)
      wb(p, b).start()

      @pl.when(p + NBUF < NSLOT)
      def _():
        wb(p, b).wait()
        gather_slot(p + NBUF, b)
    return 0
  lax.fori_loop(0, NSLOT // NBUF, group, 0)

  last = (NSLOT // NBUF - 1) * NBUF
  for b in range(NBUF):
    wb(last + b, b).wait()


def _sc_gather(srcp, x):
  f = pl.kernel(
      _sc_gather_body,
      out_type=jax.ShapeDtypeStruct((E_PAD, D), jnp.float32),
      mesh=_mesh(),
      scratch_types=[
          pltpu.VMEM((NCH, CH), jnp.int32),
          pltpu.VMEM((NBUF, SROWS, D), jnp.float32),
          pltpu.SemaphoreType.DMA((NBUF,)),
          pltpu.SemaphoreType.DMA((NBUF,)),
      ],
  )
  return f(srcp.reshape(NW, NCH, CH), x)


def _embed_body(xf, W, b, out):
  out[...] = jnp.maximum(
      jnp.dot(xf[...], W[...], preferred_element_type=jnp.float32) + b[...],
      0.0)


def _embed(edge_feat, W_ef, b_ef):
  return pl.pallas_call(
      _embed_body,
      grid=(N_EDGES // EBLK,),
      in_specs=[
          pl.BlockSpec((EBLK, DE), lambda i: (i, 0)),
          pl.BlockSpec((DE, DE), lambda i: (0, 0)),
          pl.BlockSpec((1, DE), lambda i: (0, 0)),
      ],
      out_specs=pl.BlockSpec((EBLK, DE), lambda i: (i, 0)),
      out_shape=jax.ShapeDtypeStruct((N_EDGES, DE), jnp.float32),
  )(edge_feat, W_ef, b_ef.reshape(1, DE))


def _edge_mm_body(xg, ef, W, b, out):
  m = jnp.concatenate([xg[...], ef[...]], axis=1)
  out[...] = jnp.dot(m, W[...], preferred_element_type=jnp.float32) + b[...]


def _edge_mm(ghx, ef, W, b):
  return pl.pallas_call(
      _edge_mm_body,
      grid=(N_EDGES // EBLK,),
      in_specs=[
          pl.BlockSpec((EBLK, D), lambda i: (i, 0)),
          pl.BlockSpec((EBLK, DE), lambda i: (i, 0)),
          pl.BlockSpec((D + DE, D), lambda i: (0, 0)),
          pl.BlockSpec((1, D), lambda i: (0, 0)),
      ],
      out_specs=pl.BlockSpec((EBLK, D), lambda i: (i, 0)),
      out_shape=jax.ShapeDtypeStruct((N_EDGES, D), jnp.float32),
  )(ghx, ef, W, b.reshape(1, D))


def _scatter_mean_x(vals, idx, num_segments):
  s = jax.ops.segment_sum(vals, idx, num_segments=num_segments)
  c = jax.ops.segment_sum(jnp.ones((vals.shape[0],), dtype=vals.dtype), idx,
                          num_segments=num_segments)
  c = jnp.clip(c, 1.0, None)[:, None]
  return s / c


def _graph_norm_x(x, batch, weight, bias, mean_scale):
  cnt = jax.ops.segment_sum(jnp.ones((x.shape[0],), dtype=x.dtype), batch,
                            num_segments=NSEG)
  cnt = jnp.clip(cnt, 1.0, None)[:, None]
  mean = jax.ops.segment_sum(x, batch, num_segments=NSEG) / cnt
  sub = x - mean[batch] * mean_scale
  var = jax.ops.segment_sum(sub * sub, batch, num_segments=NSEG) / cnt
  std = jnp.sqrt(var + 1e-5)
  return weight * sub / std[batch] + bias


def kernel(x, edge_index, edge_features, batch, W_ef, b_ef, W_c0, b_c0,
           W_c1, b_c1, W_c2, b_c2, gn_weight, gn_bias, gn_mean_scale,
           W_f1, b_f1, W_f2, b_f2):
  pad = E_PAD - N_EDGES
  srcp = jnp.concatenate([edge_index[0], jnp.zeros((pad,), jnp.int32)])
  dst = edge_index[1]

  ef = _embed(edge_features, W_ef, b_ef)
  h = x
  for (W, b) in ((W_c0, b_c0), (W_c1, b_c1), (W_c2, b_c2)):
    ghx = _sc_gather(srcp, h)
    he = _edge_mm(ghx, ef, W, b)
    h = _scatter_mean_x(he, dst, N_NODES)
    h = jax.nn.relu(h)
    h = _graph_norm_x(h, batch, gn_weight, gn_bias, gn_mean_scale)

  xout = _scatter_mean_x(h, batch, NSEG)
  xout = xout.reshape(-1).reshape((-1, 2 * D))
  xout1 = xout[:, :D]
  xout2 = xout[:, D:]
  xsub = xout1 - xout2
  scores = jax.nn.softmax(jax.nn.relu(xsub @ W_f1 + b_f1) @ W_f2 + b_f2,
                          axis=-1)
  return (scores.reshape(-1), h, xout1, xout2)
